# baseline (speedup 1.0000x reference)
import jax
import jax.numpy as jnp
from jax.experimental import pallas as pl
from jax.experimental.pallas import tpu as pltpu

ROWS = 1000
COLS = 65536
BLOCK_R = 8


def _body(img_ref, syn_ref, out_ref, lbl_ref, best_ref):
    i = pl.program_id(0)

    @pl.when(i == 0)
    def _init():
        best_ref[0] = -1.0
        lbl_ref[0] = 0

    feat = (img_ref[...] > 0.7).astype(jnp.float32)
    partial = jnp.sum(syn_ref[...] * feat, axis=1, keepdims=True)
    out_ref[...] = partial

    bmax = jnp.max(partial)
    idx2d = jax.lax.broadcasted_iota(jnp.int32, (BLOCK_R, 1), 0)
    local_arg = jnp.min(jnp.where(partial == bmax, idx2d, BLOCK_R))
    gidx = i * BLOCK_R + local_arg
    pred = bmax > best_ref[0]
    best_ref[0] = jnp.where(pred, bmax, best_ref[0])
    lbl_ref[0] = jnp.where(pred, gidx, lbl_ref[0])


def kernel(image, basal_synapses):
    overlaps2d, lbl = pl.pallas_call(
        _body,
        grid=(ROWS // BLOCK_R,),
        in_specs=[
            pl.BlockSpec((1, COLS), lambda i: (0, 0)),
            pl.BlockSpec((BLOCK_R, COLS), lambda i: (i, 0)),
        ],
        out_specs=[
            pl.BlockSpec((BLOCK_R, 1), lambda i: (i, 0)),
            pl.BlockSpec(memory_space=pltpu.SMEM),
        ],
        out_shape=[
            jax.ShapeDtypeStruct((ROWS, 1), jnp.float32),
            jax.ShapeDtypeStruct((1,), jnp.int32),
        ],
        scratch_shapes=[pltpu.SMEM((1,), jnp.float32)],
    )(image, basal_synapses)
    return overlaps2d.reshape(ROWS), lbl[0]


# speedup vs baseline: 1.9017x; 1.9017x over previous
import jax
import jax.numpy as jnp
from jax.experimental import pallas as pl
from jax.experimental.pallas import tpu as pltpu

ROWS = 1000
COLS = 65536
BLOCK_R = 40


def _body(img_ref, syn_ref, out_ref, lbl_ref, best_ref):
    i = pl.program_id(0)

    @pl.when(i == 0)
    def _init():
        best_ref[0] = -1.0
        lbl_ref[0] = 0

    feat = (img_ref[...] > 0.7).astype(jnp.float32)
    partial = jnp.sum(syn_ref[...] * feat, axis=1, keepdims=True)
    out_ref[...] = partial

    bmax = jnp.max(partial)
    idx2d = jax.lax.broadcasted_iota(jnp.int32, (BLOCK_R, 1), 0)
    local_arg = jnp.min(jnp.where(partial == bmax, idx2d, BLOCK_R))
    gidx = i * BLOCK_R + local_arg
    pred = bmax > best_ref[0]
    best_ref[0] = jnp.where(pred, bmax, best_ref[0])
    lbl_ref[0] = jnp.where(pred, gidx, lbl_ref[0])


def kernel(image, basal_synapses):
    overlaps2d, lbl = pl.pallas_call(
        _body,
        grid=(ROWS // BLOCK_R,),
        in_specs=[
            pl.BlockSpec((1, COLS), lambda i: (0, 0)),
            pl.BlockSpec((BLOCK_R, COLS), lambda i: (i, 0)),
        ],
        out_specs=[
            pl.BlockSpec((BLOCK_R, 1), lambda i: (i, 0)),
            pl.BlockSpec(memory_space=pltpu.SMEM),
        ],
        out_shape=[
            jax.ShapeDtypeStruct((ROWS, 1), jnp.float32),
            jax.ShapeDtypeStruct((1,), jnp.int32),
        ],
        scratch_shapes=[pltpu.SMEM((1,), jnp.float32)],
    )(image, basal_synapses)
    return overlaps2d.reshape(ROWS), lbl[0]
